# Initial kernel scaffold; baseline (speedup 1.0000x reference)
import jax
import jax.numpy as jnp
from jax.experimental import pallas as pl

Q = 256
K = 4096
D = 128
NUM_CLASSES = 1000


def _knn_kernel(x_ref, xt_ref, lab_ref, out_ref):
    x = x_ref[...]
    xt = xt_ref[...]
    labels = lab_ref[0, :]

    g = jax.lax.dot_general(
        x, xt,
        dimension_numbers=(((1,), (1,)), ((), ())),
        preferred_element_type=jnp.float32,
        precision=jax.lax.Precision.HIGHEST,
    )
    ynorm = jnp.sum(xt * xt, axis=1)
    s = ynorm[None, :] - 2.0 * g

    m = jnp.min(s, axis=1, keepdims=True)
    iota_k = jax.lax.broadcasted_iota(jnp.int32, (Q, K), 1)
    hit = s == m
    idx = jnp.min(jnp.where(hit, iota_k, K), axis=1)
    winner = iota_k == idx[:, None]
    label_q = jnp.sum(jnp.where(winner, labels[None, :], 0), axis=1)

    iota_c = jax.lax.broadcasted_iota(jnp.int32, (Q, NUM_CLASSES), 1)
    out_ref[...] = (iota_c == label_q[:, None]).astype(jnp.int32)


def kernel(x, x_train, labels_train):
    labels2d = labels_train.astype(jnp.int32).reshape(1, K)
    return pl.pallas_call(
        _knn_kernel,
        out_shape=jax.ShapeDtypeStruct((Q, NUM_CLASSES), jnp.int32),
    )(x, x_train, labels2d)


# speedup vs baseline: 11.2282x; 11.2282x over previous
import jax
import jax.numpy as jnp
from jax.experimental import pallas as pl
from jax.experimental.pallas import tpu as pltpu

Q = 256
K = 4096
D = 128
NUM_CLASSES = 1000
BK = 512
NBLK = K // BK


def _knn_kernel(x_ref, xt_ref, lab_ref, out_ref, min_ref, lab_acc_ref):
    step = pl.program_id(0)
    x = x_ref[...]
    xt = xt_ref[...]
    labels = lab_ref[...]

    g = jax.lax.dot_general(
        x, xt,
        dimension_numbers=(((1,), (1,)), ((), ())),
        preferred_element_type=jnp.float32,
        precision=jax.lax.Precision.HIGHEST,
    )
    ynorm = jax.lax.dot_general(
        jnp.ones((1, D), jnp.float32), xt * xt,
        dimension_numbers=(((1,), (1,)), ((), ())),
        preferred_element_type=jnp.float32,
        precision=jax.lax.Precision.HIGHEST,
    )
    s = ynorm - 2.0 * g

    bmin = jnp.min(s, axis=1, keepdims=True)
    iota_b = jax.lax.broadcasted_iota(jnp.int32, (Q, BK), 1)
    hit = s == bmin
    bidx = jnp.min(jnp.where(hit, iota_b, BK), axis=1, keepdims=True)
    winner = iota_b == bidx
    blabel = jnp.sum(jnp.where(winner, labels, 0), axis=1, keepdims=True)

    @pl.when(step == 0)
    def _init():
        min_ref[...] = bmin
        lab_acc_ref[...] = blabel

    @pl.when(step != 0)
    def _update():
        better = bmin < min_ref[...]
        min_ref[...] = jnp.where(better, bmin, min_ref[...])
        lab_acc_ref[...] = jnp.where(better, blabel, lab_acc_ref[...])

    @pl.when(step == NBLK - 1)
    def _emit():
        iota_c = jax.lax.broadcasted_iota(jnp.int32, (Q, NUM_CLASSES), 1)
        out_ref[...] = (iota_c == lab_acc_ref[...]).astype(jnp.int32)


def kernel(x, x_train, labels_train):
    labels2d = labels_train.astype(jnp.int32).reshape(1, K)
    return pl.pallas_call(
        _knn_kernel,
        grid=(NBLK,),
        in_specs=[
            pl.BlockSpec((Q, D), lambda i: (0, 0)),
            pl.BlockSpec((BK, D), lambda i: (i, 0)),
            pl.BlockSpec((1, BK), lambda i: (0, i)),
        ],
        out_specs=pl.BlockSpec((Q, NUM_CLASSES), lambda i: (0, 0)),
        out_shape=jax.ShapeDtypeStruct((Q, NUM_CLASSES), jnp.int32),
        scratch_shapes=[
            pltpu.VMEM((Q, 1), jnp.float32),
            pltpu.VMEM((Q, 1), jnp.int32),
        ],
    )(x, x_train, labels2d)
